# Initial kernel scaffold; baseline (speedup 1.0000x reference)
import jax
import jax.numpy as jnp
from jax.experimental import pallas as pl


def kernel(x, edge_index, w_msg, b_msg, w_ih, w_hh, b_ih, b_hh, w_up, b_up, w_down, b_down, w_pool, b_pool, w_out, b_out):
    raise NotImplementedError("write your pallas kernel here")


# speedup vs baseline: 3.7704x; 3.7704x over previous
import functools
import math

import jax
import jax.numpy as jnp
from jax import lax
from jax.experimental import pallas as pl
from jax.experimental.pallas import tpu as pltpu
from jax.experimental.pallas import tpu_sc as plsc

NC = 2
NS = 16
CHUNK = 128


def _sc_mesh():
  return plsc.VectorSubcoreMesh(
      core_axis_name="c", subcore_axis_name="s", num_cores=NC, num_subcores=NS)


def _make_scatter_add(npad, epad, width):
  cpw = epad // (NC * NS * CHUNK)
  rpt = npad // NS

  @functools.partial(
      pl.kernel,
      out_type=jax.ShapeDtypeStruct((NC * npad, width), jnp.float32),
      mesh=_sc_mesh(),
      scratch_types=[
          pltpu.VMEM((CHUNK,), jnp.int32),
          pltpu.VMEM((CHUNK,), jnp.int32),
          pltpu.VMEM((CHUNK, width), jnp.float32),
          pltpu.VMEM_SHARED((npad, width), jnp.float32),
          pltpu.SemaphoreType.DMA,
      ])
  def scatter(vals, src, dst, zrows, out, src_v, dst_v, rows_v, acc, sem):
    c = lax.axis_index("c")
    s = lax.axis_index("s")
    wid = s * NC + c
    pltpu.sync_copy(zrows, acc.at[pl.ds(s * rpt, rpt)])
    plsc.subcore_barrier()

    def step(i, carry):
      base = (wid * cpw + i) * CHUNK
      pltpu.sync_copy(src.at[pl.ds(base, CHUNK)], src_v)
      pltpu.sync_copy(dst.at[pl.ds(base, CHUNK)], dst_v)
      pltpu.async_copy(vals.at[src_v], rows_v, sem).wait()
      pltpu.sync_copy(rows_v, acc.at[dst_v], add=True)
      return carry

    lax.fori_loop(0, cpw, step, 0)
    plsc.subcore_barrier()
    pltpu.sync_copy(acc.at[pl.ds(s * rpt, rpt)],
                    out.at[pl.ds(c * npad + s * rpt, rpt)])

  return scatter


def _make_count(npad, epad):
  cpw = epad // (NC * NS * CHUNK)
  rpt = npad // NS

  @functools.partial(
      pl.kernel,
      out_type=jax.ShapeDtypeStruct((NC * npad, 128), jnp.float32),
      mesh=_sc_mesh(),
      scratch_types=[
          pltpu.VMEM((CHUNK,), jnp.int32),
          pltpu.VMEM((CHUNK, 128), jnp.float32),
          pltpu.VMEM_SHARED((npad, 128), jnp.float32),
      ])
  def count(idx, ones_rows, zrows, out, idx_v, ones_v, acc):
    c = lax.axis_index("c")
    s = lax.axis_index("s")
    wid = s * NC + c
    pltpu.sync_copy(ones_rows, ones_v)
    pltpu.sync_copy(zrows, acc.at[pl.ds(s * rpt, rpt)])
    plsc.subcore_barrier()

    def step(i, carry):
      base = (wid * cpw + i) * CHUNK
      pltpu.sync_copy(idx.at[pl.ds(base, CHUNK)], idx_v)
      pltpu.sync_copy(ones_v, acc.at[idx_v], add=True)
      return carry

    lax.fori_loop(0, cpw, step, 0)
    plsc.subcore_barrier()
    pltpu.sync_copy(acc.at[pl.ds(s * rpt, rpt)],
                    out.at[pl.ds(c * npad + s * rpt, rpt)])

  return count


_ROWS = 512


def _row_spec(width):
  return pl.BlockSpec((_ROWS, width), lambda i: (i, 0))


def _full_spec(shape):
  return pl.BlockSpec(shape, lambda i: tuple(0 for _ in shape))


def _tc_prep(do0, do1, di0, di1, b_msg, npad):
  def body(do0_r, do1_r, di0_r, di1_r, bm_r, bmsg_o, din_o, dout_o):
    deg_o = do0_r[:, 0:1] + do1_r[:, 0:1]
    deg_i = di0_r[:, 0:1] + di1_r[:, 0:1]
    bmsg_o[...] = deg_i * bm_r[...]
    shape = din_o.shape
    din_o[...] = jnp.broadcast_to(lax.rsqrt(jnp.maximum(deg_i, 1.0)), shape)
    dout_o[...] = jnp.broadcast_to(lax.rsqrt(jnp.maximum(deg_o, 1.0)), shape)

  out_sh = jax.ShapeDtypeStruct((npad, 128), jnp.float32)
  return pl.pallas_call(
      body,
      grid=(npad // _ROWS,),
      in_specs=[_row_spec(128)] * 4 + [_full_spec((1, 128))],
      out_specs=[_row_spec(128)] * 3,
      out_shape=[out_sh] * 3,
  )(do0, do1, di0, di1, b_msg)


def _tc_gru_round(s_parts, bmsg, h, w_msg, w_ih, w_hh, b_ih, b_hh, dinv_out,
                  npad):
  def body(s0_r, s1_r, bmsg_r, h_r, wm_r, wih_r, whh_r, bih_r, bhh_r, do_r,
           hn_o, hh_o):
    S = s0_r[...] + s1_r[...]
    a = lax.dot_general(S, wm_r[...], (((1,), (1,)), ((), ()))) + bmsg_r[...]
    h = h_r[...]
    gi = lax.dot_general(a, wih_r[...], (((1,), (1,)), ((), ()))) + bih_r[...]
    gh = lax.dot_general(h, whh_r[...], (((1,), (1,)), ((), ()))) + bhh_r[...]
    r = jax.nn.sigmoid(gi[:, 0:128] + gh[:, 0:128])
    z = jax.nn.sigmoid(gi[:, 128:256] + gh[:, 128:256])
    n = jnp.tanh(gi[:, 256:384] + r * gh[:, 256:384])
    hn = (1.0 - z) * n + z * h
    hn_o[...] = hn
    hh_o[...] = hn * do_r[...]

  out_sh = jax.ShapeDtypeStruct((npad, 128), jnp.float32)
  return pl.pallas_call(
      body,
      grid=(npad // _ROWS,),
      in_specs=[_row_spec(128)] * 4 + [
          _full_spec((128, 128)), _full_spec((384, 128)),
          _full_spec((384, 128)), _full_spec((1, 384)), _full_spec((1, 384)),
          _row_spec(128),
      ],
      out_specs=[_row_spec(128)] * 2,
      out_shape=[out_sh] * 2,
  )(s_parts[0], s_parts[1], bmsg, h, w_msg, w_ih, w_hh, b_ih, b_hh, dinv_out)


def _tc_upconv(sh_parts, dinv_in, dinv_out, w_up, b_up, w_down_p, w_pool_p,
               b_pool_p, npad, n_real):
  nblk = npad // _ROWS

  def body(s0_r, s1_r, din_r, do_r, wup_r, bup_r, wd_r, wp_r, bp_r,
           xu_o, y_o, g_o):
    i = pl.program_id(0)
    agg = (s0_r[...] + s1_r[...]) * din_r[...]
    xu = lax.dot_general(agg, wup_r[...], (((1,), (0,)), ((), ()))) + bup_r[...]
    xu = jnp.where(xu >= 0.0, xu, 0.01 * xu)
    xu_o[...] = xu
    y_o[...] = do_r[:, 0:1] * lax.dot_general(
        xu, wd_r[...], (((1,), (0,)), ((), ())))
    g = lax.dot_general(xu, wp_r[...], (((1,), (0,)), ((), ()))) + bp_r[...]
    rows = i * _ROWS + lax.broadcasted_iota(jnp.int32, g.shape, 0)
    g_o[...] = jnp.where(rows < n_real, g, -1e30)

  return pl.pallas_call(
      body,
      grid=(nblk,),
      in_specs=[_row_spec(128)] * 4 + [
          _full_spec((128, 1024)), _full_spec((1, 1024)),
          _full_spec((1024, 128)), _full_spec((1024, 16)),
          _full_spec((1, 16)),
      ],
      out_specs=[_row_spec(1024), _row_spec(128), _row_spec(16)],
      out_shape=[
          jax.ShapeDtypeStruct((npad, 1024), jnp.float32),
          jax.ShapeDtypeStruct((npad, 128), jnp.float32),
          jax.ShapeDtypeStruct((npad, 16), jnp.float32),
      ],
  )(sh_parts[0], sh_parts[1], dinv_in, dinv_out, w_up, b_up, w_down_p,
    w_pool_p, b_pool_p)


def _tc_gmax(g, npad):
  def body(g_r, out_r):
    i = pl.program_id(0)

    @pl.when(i == 0)
    def _():
      out_r[...] = jnp.full_like(out_r, -3e38)

    out_r[...] = jnp.maximum(out_r[...], jnp.max(g_r[...], axis=0,
                                                 keepdims=True))

  return pl.pallas_call(
      body,
      grid=(npad // _ROWS,),
      in_specs=[_row_spec(16)],
      out_specs=pl.BlockSpec((1, 16), lambda i: (0, 0)),
      out_shape=jax.ShapeDtypeStruct((1, 16), jnp.float32),
  )(g)


def _tc_final(sy_parts, dinv_in, g, gmax, xu, b_down_p, w_out_p, b_out_p,
              npad):
  nblk = npad // _ROWS

  def body(sy0_r, sy1_r, din_r, g_r, gmax_r, xu_r, bd_r, wo_r, bo_r,
           colors_o, z_o, p_o, sat_o):
    i = pl.program_id(0)

    @pl.when(i == 0)
    def _():
      z_o[...] = jnp.zeros_like(z_o)
      p_o[...] = jnp.zeros_like(p_o)

    xd = (sy0_r[...] + sy1_r[...]) * din_r[:, 0:1] + bd_r[...]
    cols = lax.broadcasted_iota(jnp.int32, xd.shape, 1)
    xdm = jnp.where(cols < 3, xd, -3e38)
    m = jnp.max(xdm, axis=1, keepdims=True)
    e = jnp.where(cols < 3, jnp.exp(xdm - m), 0.0)
    colors_o[...] = e / jnp.sum(e, axis=1, keepdims=True)

    w = jnp.exp(g_r[...] - gmax_r[...])
    z_o[...] += jnp.sum(w, axis=0, keepdims=True)
    p_o[...] += jnp.sum(w[:, 0:1] * xu_r[...], axis=0, keepdims=True)

    @pl.when(i == nblk - 1)
    def _():
      pooled = p_o[...] / z_o[0:1, 0:1]
      s = lax.dot_general(pooled, wo_r[...], (((1,), (0,)), ((), ())))
      sat_o[...] = jax.nn.sigmoid(s + bo_r[...])

  return pl.pallas_call(
      body,
      grid=(nblk,),
      in_specs=[_row_spec(128), _row_spec(128), _row_spec(128), _row_spec(16),
                _full_spec((1, 16)), _row_spec(1024), _full_spec((1, 128)),
                _full_spec((1024, 16)), _full_spec((1, 16))],
      out_specs=[_row_spec(128),
                 pl.BlockSpec((1, 16), lambda i: (0, 0)),
                 pl.BlockSpec((1, 1024), lambda i: (0, 0)),
                 pl.BlockSpec((1, 16), lambda i: (0, 0))],
      out_shape=[
          jax.ShapeDtypeStruct((npad, 128), jnp.float32),
          jax.ShapeDtypeStruct((1, 16), jnp.float32),
          jax.ShapeDtypeStruct((1, 1024), jnp.float32),
          jax.ShapeDtypeStruct((1, 16), jnp.float32),
      ],
  )(sy_parts[0], sy_parts[1], dinv_in, g, gmax, xu, b_down_p, w_out_p,
    b_out_p)


def kernel(x, edge_index, w_msg, b_msg, w_ih, w_hh, b_ih, b_hh, w_up, b_up,
           w_down, b_down, w_pool, b_pool, w_out, b_out):
  n, hdim = x.shape
  e = edge_index.shape[1]
  T = 5

  grain = NC * NS * CHUNK
  epad = math.ceil(e / grain) * grain
  npad = math.ceil((n + 1) / (NS * CHUNK)) * (NS * CHUNK)
  rpt = npad // NS

  pad_e = epad - e
  src = jnp.concatenate([edge_index[0], jnp.full((pad_e,), n, jnp.int32)])
  dst = jnp.concatenate([edge_index[1], jnp.full((pad_e,), n, jnp.int32)])

  z128 = jnp.zeros((rpt, 128), jnp.float32)
  ones128 = jnp.ones((CHUNK, 128), jnp.float32)

  scat128 = _make_scatter_add(npad, epad, 128)
  count = _make_count(npad, epad)

  deg_o = count(src, ones128, z128)
  deg_i = count(dst, ones128, z128)
  bmsg_eff, dinv_in, dinv_out = _tc_prep(
      deg_o[:npad], deg_o[npad:], deg_i[:npad], deg_i[npad:],
      b_msg.reshape(1, 128), npad)

  h = jnp.pad(x, ((0, npad - n), (0, 0))) * (hdim ** -0.5)
  hh = None
  for _ in range(T):
    sp = scat128(h, src, dst, z128)
    h, hh = _tc_gru_round((sp[:npad], sp[npad:]), bmsg_eff, h, w_msg, w_ih,
                          w_hh, b_ih.reshape(1, 384), b_hh.reshape(1, 384),
                          dinv_out, npad)

  shp = scat128(hh, src, dst, z128)
  w_down_p = jnp.pad(w_down, ((0, 0), (0, 128 - w_down.shape[1])))
  w_pool_p = jnp.broadcast_to(w_pool, (1024, 16))
  b_pool_p = jnp.broadcast_to(b_pool.reshape(1, 1), (1, 16))
  xu, y, g = _tc_upconv((shp[:npad], shp[npad:]), dinv_in, dinv_out, w_up,
                        b_up.reshape(1, 1024), w_down_p, w_pool_p, b_pool_p,
                        npad, n)

  syp = scat128(y, src, dst, z128)
  gmax = _tc_gmax(g, npad)
  b_down_p = jnp.pad(b_down, (0, 125)).reshape(1, 128)
  w_out_p = jnp.broadcast_to(w_out, (1024, 16))
  b_out_p = jnp.broadcast_to(b_out.reshape(1, 1), (1, 16))
  colors_p, _, _, sat_p = _tc_final((syp[:npad], syp[npad:]), dinv_in, g,
                                    gmax, xu, b_down_p, w_out_p, b_out_p,
                                    npad)

  colors = colors_p[:n, :3]
  sat = sat_p[0, 0]
  return (colors, sat)


# speedup vs baseline: 11.1805x; 2.9653x over previous
import functools
import math

import jax
import jax.numpy as jnp
from jax import lax
from jax.experimental import pallas as pl
from jax.experimental.pallas import tpu as pltpu
from jax.experimental.pallas import tpu_sc as plsc

NC = 2
NS = 16
CHUNK = 128


def _sc_mesh():
  return plsc.VectorSubcoreMesh(
      core_axis_name="c", subcore_axis_name="s", num_cores=NC, num_subcores=NS)


BATCH = 8


def _make_scatter_add(npad, epad, width):
  cpw = epad // (NC * NS * CHUNK)
  nb = cpw // BATCH
  assert cpw % BATCH == 0 and nb % 2 == 0 and nb >= 2
  rpt = npad // NS

  @functools.partial(
      pl.kernel,
      out_type=jax.ShapeDtypeStruct((NC * npad, width), jnp.float32),
      mesh=_sc_mesh(),
      scratch_types=[
          pltpu.VMEM((2 * BATCH, CHUNK), jnp.int32),
          pltpu.VMEM((2 * BATCH, CHUNK), jnp.int32),
          pltpu.VMEM((CHUNK, width), jnp.float32),
          pltpu.VMEM((CHUNK, width), jnp.float32),
          pltpu.VMEM_SHARED((npad, width), jnp.float32),
          pltpu.SemaphoreType.DMA,
          pltpu.SemaphoreType.DMA,
      ])
  def scatter(vals, pidx, zrows, out, idx0, idx1, rows0, rows1, acc,
              gsem0, gsem1):
    c = lax.axis_index("c")
    s = lax.axis_index("s")
    wid = s * NC + c
    idxb, rowsb, gsem = (idx0, idx1), (rows0, rows1), (gsem0, gsem1)
    base_b = wid * nb

    pltpu.sync_copy(zrows.at[pl.ds(0, rpt)], acc.at[pl.ds(s * rpt, rpt)])
    plsc.subcore_barrier()

    def load_idx(p, b):
      pltpu.sync_copy(pidx.at[pl.ds((base_b + b) * 2 * BATCH, 2 * BATCH)],
                      idxb[p])

    def fire(p, j):
      pltpu.async_copy(vals.at[idxb[p].at[2 * j]], rowsb[j % 2],
                       gsem[j % 2])

    def drain_scatter(p, j):
      pltpu.make_async_copy(vals.at[pl.ds(0, CHUNK)], rowsb[j % 2],
                            gsem[j % 2]).wait()
      pltpu.sync_copy(rowsb[j % 2], acc.at[idxb[p].at[2 * j + 1]], add=True)

    load_idx(0, 0)
    fire(0, 0)

    def pair(i, carry):
      for half in (0, 1):
        b = 2 * i + half

        @pl.when(b + 1 < nb)
        def _():
          load_idx(1 - half, b + 1)

        for j in range(BATCH):
          if j + 1 < BATCH:
            fire(half, j + 1)
          else:
            @pl.when(b + 1 < nb)
            def _():
              fire(1 - half, 0)
          drain_scatter(half, j)
      return carry

    lax.fori_loop(0, nb // 2, pair, 0)
    plsc.subcore_barrier()
    pltpu.sync_copy(acc.at[pl.ds(s * rpt, rpt)],
                    out.at[pl.ds(c * npad + s * rpt, rpt)])

  return scatter


def _make_count(npad, epad):
  B = 8
  nchunks = epad // CHUNK
  cpt = nchunks // NS
  bpt = cpt // B
  assert cpt % B == 0 and bpt % 2 == 0 and bpt >= 4
  rpt = npad // NS

  @functools.partial(
      pl.kernel,
      out_type=jax.ShapeDtypeStruct((NC * npad, 128), jnp.float32),
      mesh=_sc_mesh(),
      scratch_types=[
          pltpu.VMEM((B, CHUNK), jnp.int32),
          pltpu.VMEM((CHUNK, 128), jnp.float32),
          pltpu.VMEM_SHARED((npad, 128), jnp.float32),
      ])
  def count(cidx, ones_rows, zrows, out, idx_v, ones_v, acc):
    c = lax.axis_index("c")
    s = lax.axis_index("s")
    base_b = (c * NS + s) * bpt

    pltpu.sync_copy(ones_rows, ones_v)
    pltpu.sync_copy(zrows.at[pl.ds(0, rpt)], acc.at[pl.ds(s * rpt, rpt)])
    plsc.subcore_barrier()

    def step(g, carry):
      pltpu.sync_copy(cidx.at[pl.ds((base_b + g) * B, B)], idx_v)
      for j in range(B):
        pltpu.sync_copy(ones_v, acc.at[idx_v.at[j]], add=True)
      return carry

    lax.fori_loop(0, bpt, step, 0)
    plsc.subcore_barrier()
    pltpu.sync_copy(acc.at[pl.ds(s * rpt, rpt)],
                    out.at[pl.ds(c * npad + s * rpt, rpt)])

  return count


_ROWS = 512


def _row_spec(width):
  return pl.BlockSpec((_ROWS, width), lambda i: (i, 0))


def _full_spec(shape):
  return pl.BlockSpec(shape, lambda i: tuple(0 for _ in shape))


def _tc_prep(do, di, b_msg, npad):
  def body(do_r, di_r, bm_r, bmsg_o, din_o, dout_o):
    deg_o = do_r[:, 0:1]
    deg_i = di_r[:, 0:1]
    bmsg_o[...] = deg_i * bm_r[...]
    shape = din_o.shape
    din_o[...] = jnp.broadcast_to(lax.rsqrt(jnp.maximum(deg_i, 1.0)), shape)
    dout_o[...] = jnp.broadcast_to(lax.rsqrt(jnp.maximum(deg_o, 1.0)), shape)

  out_sh = jax.ShapeDtypeStruct((npad, 128), jnp.float32)
  return pl.pallas_call(
      body,
      grid=(npad // _ROWS,),
      in_specs=[_row_spec(128)] * 2 + [_full_spec((1, 128))],
      out_specs=[_row_spec(128)] * 3,
      out_shape=[out_sh] * 3,
  )(do, di, b_msg)


def _tc_gru_round(s_parts, bmsg, h, w_msg, w_ih, w_hh, b_ih, b_hh, dinv_out,
                  npad):
  def body(s0_r, s1_r, bmsg_r, h_r, wm_r, wih_r, whh_r, bih_r, bhh_r, do_r,
           hn_o, hh_o):
    S = s0_r[...] + s1_r[...]
    a = lax.dot_general(S, wm_r[...], (((1,), (1,)), ((), ()))) + bmsg_r[...]
    h = h_r[...]
    gi = lax.dot_general(a, wih_r[...], (((1,), (1,)), ((), ()))) + bih_r[...]
    gh = lax.dot_general(h, whh_r[...], (((1,), (1,)), ((), ()))) + bhh_r[...]
    r = jax.nn.sigmoid(gi[:, 0:128] + gh[:, 0:128])
    z = jax.nn.sigmoid(gi[:, 128:256] + gh[:, 128:256])
    n = jnp.tanh(gi[:, 256:384] + r * gh[:, 256:384])
    hn = (1.0 - z) * n + z * h
    hn_o[...] = hn
    hh_o[...] = hn * do_r[...]

  out_sh = jax.ShapeDtypeStruct((npad, 128), jnp.float32)
  return pl.pallas_call(
      body,
      grid=(npad // _ROWS,),
      in_specs=[_row_spec(128)] * 4 + [
          _full_spec((128, 128)), _full_spec((384, 128)),
          _full_spec((384, 128)), _full_spec((1, 384)), _full_spec((1, 384)),
          _row_spec(128),
      ],
      out_specs=[_row_spec(128)] * 2,
      out_shape=[out_sh] * 2,
  )(s_parts[0], s_parts[1], bmsg, h, w_msg, w_ih, w_hh, b_ih, b_hh, dinv_out)


def _tc_upconv(sh_parts, dinv_in, dinv_out, w_up, b_up, w_down_p, w_pool_p,
               b_pool_p, npad, n_real):
  nblk = npad // _ROWS

  def body(s0_r, s1_r, din_r, do_r, wup_r, bup_r, wd_r, wp_r, bp_r,
           xu_o, y_o, g_o):
    i = pl.program_id(0)
    agg = (s0_r[...] + s1_r[...]) * din_r[...]
    xu = lax.dot_general(agg, wup_r[...], (((1,), (0,)), ((), ()))) + bup_r[...]
    xu = jnp.where(xu >= 0.0, xu, 0.01 * xu)
    xu_o[...] = xu
    y_o[...] = do_r[:, 0:1] * lax.dot_general(
        xu, wd_r[...], (((1,), (0,)), ((), ())))
    g = lax.dot_general(xu, wp_r[...], (((1,), (0,)), ((), ()))) + bp_r[...]
    rows = i * _ROWS + lax.broadcasted_iota(jnp.int32, g.shape, 0)
    g_o[...] = jnp.where(rows < n_real, g, -1e30)

  return pl.pallas_call(
      body,
      grid=(nblk,),
      in_specs=[_row_spec(128)] * 4 + [
          _full_spec((128, 1024)), _full_spec((1, 1024)),
          _full_spec((1024, 128)), _full_spec((1024, 16)),
          _full_spec((1, 16)),
      ],
      out_specs=[_row_spec(1024), _row_spec(128), _row_spec(16)],
      out_shape=[
          jax.ShapeDtypeStruct((npad, 1024), jnp.float32),
          jax.ShapeDtypeStruct((npad, 128), jnp.float32),
          jax.ShapeDtypeStruct((npad, 16), jnp.float32),
      ],
  )(sh_parts[0], sh_parts[1], dinv_in, dinv_out, w_up, b_up, w_down_p,
    w_pool_p, b_pool_p)


def _tc_gmax(g, npad):
  def body(g_r, out_r):
    i = pl.program_id(0)

    @pl.when(i == 0)
    def _():
      out_r[...] = jnp.full_like(out_r, -3e38)

    out_r[...] = jnp.maximum(out_r[...], jnp.max(g_r[...], axis=0,
                                                 keepdims=True))

  return pl.pallas_call(
      body,
      grid=(npad // _ROWS,),
      in_specs=[_row_spec(16)],
      out_specs=pl.BlockSpec((1, 16), lambda i: (0, 0)),
      out_shape=jax.ShapeDtypeStruct((1, 16), jnp.float32),
  )(g)


def _tc_final(sy_parts, dinv_in, g, gmax, xu, b_down_p, w_out_p, b_out_p,
              npad):
  nblk = npad // _ROWS

  def body(sy0_r, sy1_r, din_r, g_r, gmax_r, xu_r, bd_r, wo_r, bo_r,
           colors_o, z_o, p_o, sat_o):
    i = pl.program_id(0)

    @pl.when(i == 0)
    def _():
      z_o[...] = jnp.zeros_like(z_o)
      p_o[...] = jnp.zeros_like(p_o)

    xd = (sy0_r[...] + sy1_r[...]) * din_r[:, 0:1] + bd_r[...]
    cols = lax.broadcasted_iota(jnp.int32, xd.shape, 1)
    xdm = jnp.where(cols < 3, xd, -3e38)
    m = jnp.max(xdm, axis=1, keepdims=True)
    e = jnp.where(cols < 3, jnp.exp(xdm - m), 0.0)
    colors_o[...] = e / jnp.sum(e, axis=1, keepdims=True)

    w = jnp.exp(g_r[...] - gmax_r[...])
    z_o[...] += jnp.sum(w, axis=0, keepdims=True)
    p_o[...] += jnp.sum(w[:, 0:1] * xu_r[...], axis=0, keepdims=True)

    @pl.when(i == nblk - 1)
    def _():
      pooled = p_o[...] / z_o[0:1, 0:1]
      s = lax.dot_general(pooled, wo_r[...], (((1,), (0,)), ((), ())))
      sat_o[...] = jax.nn.sigmoid(s + bo_r[...])

  return pl.pallas_call(
      body,
      grid=(nblk,),
      in_specs=[_row_spec(128), _row_spec(128), _row_spec(128), _row_spec(16),
                _full_spec((1, 16)), _row_spec(1024), _full_spec((1, 128)),
                _full_spec((1024, 16)), _full_spec((1, 16))],
      out_specs=[_row_spec(128),
                 pl.BlockSpec((1, 16), lambda i: (0, 0)),
                 pl.BlockSpec((1, 1024), lambda i: (0, 0)),
                 pl.BlockSpec((1, 16), lambda i: (0, 0))],
      out_shape=[
          jax.ShapeDtypeStruct((npad, 128), jnp.float32),
          jax.ShapeDtypeStruct((1, 16), jnp.float32),
          jax.ShapeDtypeStruct((1, 1024), jnp.float32),
          jax.ShapeDtypeStruct((1, 16), jnp.float32),
      ],
  )(sy_parts[0], sy_parts[1], dinv_in, g, gmax, xu, b_down_p, w_out_p,
    b_out_p)


def kernel(x, edge_index, w_msg, b_msg, w_ih, w_hh, b_ih, b_hh, w_up, b_up,
           w_down, b_down, w_pool, b_pool, w_out, b_out):
  n, hdim = x.shape
  e = edge_index.shape[1]
  T = 5

  grain = NC * NS * CHUNK * 2 * BATCH
  epad = math.ceil(e / grain) * grain
  npad = math.ceil((n + 1) / (NS * CHUNK)) * (NS * CHUNK)

  pad_e = epad - e
  pad_idx = (n + 1 + jnp.arange(pad_e, dtype=jnp.int32)
             % jnp.int32(npad - n - 1))
  src = jnp.concatenate([edge_index[0], pad_idx])
  dst = jnp.concatenate([edge_index[1], pad_idx])
  es = src.reshape(-1, CHUNK)
  ed = dst.reshape(-1, CHUNK)
  pidx = jnp.stack([es, ed], axis=1).reshape(-1, CHUNK)
  cidx = jnp.concatenate([es, ed], axis=0)

  zbig = jnp.zeros((1024, 128), jnp.float32)
  ones128 = jnp.ones((CHUNK, 128), jnp.float32)

  scat128 = _make_scatter_add(npad, epad, 128)
  count = _make_count(npad, epad)

  deg = count(cidx, ones128, zbig)
  bmsg_eff, dinv_in, dinv_out = _tc_prep(
      deg[:npad], deg[npad:], b_msg.reshape(1, 128), npad)

  h = jnp.pad(x, ((0, npad - n), (0, 0))) * (hdim ** -0.5)
  hh = None
  for _ in range(T):
    sp = scat128(h, pidx, zbig)
    h, hh = _tc_gru_round((sp[:npad], sp[npad:]), bmsg_eff, h, w_msg, w_ih,
                          w_hh, b_ih.reshape(1, 384), b_hh.reshape(1, 384),
                          dinv_out, npad)

  shp = scat128(hh, pidx, zbig)
  w_down_p = jnp.pad(w_down, ((0, 0), (0, 128 - w_down.shape[1])))
  w_pool_p = jnp.broadcast_to(w_pool, (1024, 16))
  b_pool_p = jnp.broadcast_to(b_pool.reshape(1, 1), (1, 16))
  xu, y, g = _tc_upconv((shp[:npad], shp[npad:]), dinv_in, dinv_out, w_up,
                        b_up.reshape(1, 1024), w_down_p, w_pool_p, b_pool_p,
                        npad, n)

  syp = scat128(y, pidx, zbig)
  gmax = _tc_gmax(g, npad)
  b_down_p = jnp.pad(b_down, (0, 125)).reshape(1, 128)
  w_out_p = jnp.broadcast_to(w_out, (1024, 16))
  b_out_p = jnp.broadcast_to(b_out.reshape(1, 1), (1, 16))
  colors_p, _, _, sat_p = _tc_final((syp[:npad], syp[npad:]), dinv_in, g,
                                    gmax, xu, b_down_p, w_out_p, b_out_p,
                                    npad)

  colors = colors_p[:n, :3]
  sat = sat_p[0, 0]
  return (colors, sat)
